# Initial kernel scaffold; baseline (speedup 1.0000x reference)
import jax
import jax.numpy as jnp
from jax.experimental import pallas as pl


def kernel(tokens, word2state, state_emb, next_state_emb, projection, start_emb):
    raise NotImplementedError("write your pallas kernel here")


# speedup vs baseline: 1.7397x; 1.7397x over previous
import functools

import jax
import jax.numpy as jnp
from jax import lax
from jax.experimental import pallas as pl
from jax.experimental.pallas import tpu as pltpu
from jax.experimental.pallas import tpu_sc as plsc

_NC = 2
_NS = 16
_NW = _NC * _NS
_LANES = 16


def _log_softmax(x):
    m = jnp.max(x, axis=-1, keepdims=True)
    s = jnp.sum(jnp.exp(x - m), axis=-1, keepdims=True)
    return x - m - jnp.log(s)


def _precompute_body(se_ref, ne_ref, proj_ref, start_ref, bp_ref, cp_ref,
                     ap_ref):
    se = se_ref[...]
    ne = ne_ref[...]
    proj = proj_ref[...]
    start = start_ref[...]
    bp_ref[...] = _log_softmax(
        jnp.dot(se, proj, preferred_element_type=jnp.float32))
    cp_ref[...] = _log_softmax(
        jnp.dot(ne, proj, preferred_element_type=jnp.float32))
    ap = jnp.dot(se, start, preferred_element_type=jnp.float32)
    ap_ref[...] = jnp.broadcast_to(ap, (ap.shape[0], _LANES))


def _precompute(state_emb, next_state_emb, projection, start_emb):
    c, h = state_emb.shape
    d = projection.shape[1]
    blk = 2048
    grid = c // blk
    return pl.pallas_call(
        _precompute_body,
        grid=(grid,),
        in_specs=[
            pl.BlockSpec((blk, h), lambda i: (i, 0)),
            pl.BlockSpec((blk, h), lambda i: (i, 0)),
            pl.BlockSpec((h, d), lambda i: (0, 0)),
            pl.BlockSpec((h, 1), lambda i: (0, 0)),
        ],
        out_specs=[
            pl.BlockSpec((blk, d), lambda i: (i, 0)),
            pl.BlockSpec((blk, d), lambda i: (i, 0)),
            pl.BlockSpec((blk, _LANES), lambda i: (i, 0)),
        ],
        out_shape=[
            jax.ShapeDtypeStruct((c, d), jnp.float32),
            jax.ShapeDtypeStruct((c, d), jnp.float32),
            jax.ShapeDtypeStruct((c, _LANES), jnp.float32),
        ],
    )(state_emb, next_state_emb, projection, start_emb.reshape(h, 1))


def _make_sc_gather(ntok, s, d, c):
    tpw = ntok // _NW
    mesh = plsc.VectorSubcoreMesh(core_axis_name="c", subcore_axis_name="s")

    @functools.partial(
        pl.kernel,
        out_type=(
            jax.ShapeDtypeStruct((ntok, s, d), jnp.float32),
            jax.ShapeDtypeStruct((ntok, s, d), jnp.float32),
            jax.ShapeDtypeStruct((ntok, s, _LANES), jnp.float32),
            jax.ShapeDtypeStruct((_NC, c, _LANES), jnp.float32),
        ),
        mesh=mesh,
        scratch_types=[
            pltpu.VMEM((tpw,), jnp.int32),
            pltpu.VMEM((tpw, s), jnp.int32),
            pltpu.VMEM((tpw, s, d), jnp.float32),
            pltpu.VMEM((tpw, s, d), jnp.float32),
            pltpu.VMEM((tpw, s, _LANES), jnp.float32),
            pltpu.VMEM((tpw, s, _LANES), jnp.float32),
            pltpu.VMEM_SHARED((c, _LANES), jnp.float32),
            pltpu.SemaphoreType.DMA,
            pltpu.SemaphoreType.DMA,
            pltpu.SemaphoreType.DMA,
        ],
        compiler_params=pltpu.CompilerParams(use_tc_tiling_on_sc=False),
    )
    def sc_gather(tok_hbm, w2s_hbm, bp_hbm, cp_hbm, ap_hbm, zero_hbm,
                  bg_out, cg_out, ag_out, mem_out,
                  tok_v, st_v, bg_v, cg_v, ag_v, ea_v, mem_sh,
                  sem_b, sem_c, sem_a):
        cid = lax.axis_index("c")
        sid = lax.axis_index("s")
        wid = sid * _NC + cid
        base = wid * tpw

        @pl.when(sid == 0)
        def _():
            pltpu.sync_copy(zero_hbm, mem_sh)

        pltpu.sync_copy(tok_hbm.at[pl.ds(base, tpw)], tok_v)
        pltpu.async_copy(w2s_hbm.at[tok_v], st_v, sem_b).wait()

        descs = []
        for i in range(tpw):
            descs.append(pltpu.async_copy(bp_hbm.at[st_v.at[i]],
                                          bg_v.at[i], sem_b))
            descs.append(pltpu.async_copy(cp_hbm.at[st_v.at[i]],
                                          cg_v.at[i], sem_c))
        ag_descs = []
        for i in range(tpw):
            ag_descs.append(pltpu.async_copy(ap_hbm.at[st_v.at[i]],
                                             ag_v.at[i], sem_a))
        for de in ag_descs:
            de.wait()

        neg_inf = jnp.full((_LANES,), -jnp.inf, jnp.float32)
        for i in range(tpw):
            m = lax.fori_loop(
                0, s, lambda j, acc: jnp.maximum(acc, ag_v[i, j]), neg_inf)

            def zbody(j, acc, i=i, m=m):
                eu = jnp.exp(ag_v[i, j] - m)
                ea_v[i, j] = eu
                return acc + eu

            z = lax.fori_loop(0, s, zbody, jnp.zeros((_LANES,), jnp.float32))
            rz = jnp.float32(1.0) / z

            def sbody(j, acc, i=i, rz=rz):
                ea_v[i, j] = ea_v[i, j] * rz
                return acc

            lax.fori_loop(0, s, sbody, jnp.int32(0))

        for de in descs:
            de.wait()

        pltpu.sync_copy(bg_v, bg_out.at[pl.ds(base, tpw)])
        pltpu.sync_copy(cg_v, cg_out.at[pl.ds(base, tpw)])
        pltpu.sync_copy(ag_v, ag_out.at[pl.ds(base, tpw)])

        plsc.subcore_barrier()
        for i in range(tpw):
            pltpu.sync_copy(ea_v.at[i], mem_sh.at[st_v.at[i]], add=True)
        plsc.subcore_barrier()

        @pl.when((sid == 0) & (cid == 0))
        def _():
            pltpu.sync_copy(mem_sh, mem_out.at[0])

        @pl.when((sid == 0) & (cid == 1))
        def _():
            pltpu.sync_copy(mem_sh, mem_out.at[1])

    return sc_gather


def _logmvv_body(bg_ref, cg_ref, ag_ref, memp_ref, out_ref, mem_ref):
    tb, s, d = bg_ref.shape
    ag = ag_ref[:, :, 0]
    a = _log_softmax(ag)
    m_a = jnp.max(a, axis=-1, keepdims=True)
    ea = jnp.exp(a - m_a)
    bg = bg_ref[...]
    cg = cg_ref[...]
    m_b = jnp.max(bg, axis=1, keepdims=True)
    m_c = jnp.max(cg, axis=1, keepdims=True)
    eb = jnp.exp(bg - m_b)
    ec = jnp.exp(cg - m_c)
    w = ea[:, :, None] * eb
    sm = lax.dot_general(w, ec,
                         dimension_numbers=(((1,), (1,)), ((0,), (0,))),
                         preferred_element_type=jnp.float32)
    out_ref[...] = (m_a[:, :, None] + m_b.reshape(tb, d, 1)
                    + m_c.reshape(tb, 1, d) + jnp.log(sm + 1e-30))
    mem_ref[...] = (memp_ref[0, :, 0] + memp_ref[1, :, 0])[None, None, :]


def _logmvv(bg, cg, ag, memp):
    ntok, s, d = bg.shape
    c = memp.shape[1]
    tb = 8
    grid = ntok // tb
    cblk = c // grid
    out, mem = pl.pallas_call(
        _logmvv_body,
        grid=(grid,),
        in_specs=[
            pl.BlockSpec((tb, s, d), lambda i: (i, 0, 0)),
            pl.BlockSpec((tb, s, d), lambda i: (i, 0, 0)),
            pl.BlockSpec((tb, s, _LANES), lambda i: (i, 0, 0)),
            pl.BlockSpec((_NC, cblk, _LANES), lambda i: (0, i, 0)),
        ],
        out_specs=[
            pl.BlockSpec((tb, d, d), lambda i: (i, 0, 0)),
            pl.BlockSpec((1, 1, cblk), lambda i: (i, 0, 0)),
        ],
        out_shape=[
            jax.ShapeDtypeStruct((ntok, d, d), jnp.float32),
            jax.ShapeDtypeStruct((grid, 1, cblk), jnp.float32),
        ],
    )(bg, cg, ag, memp)
    return out, mem.reshape(c)


def kernel(tokens, word2state, state_emb, next_state_emb, projection,
           start_emb):
    n, t = tokens.shape
    c, h = state_emb.shape
    s = word2state.shape[1]
    d = projection.shape[1]
    ntok = n * t

    bp, cp, ap = _precompute(state_emb, next_state_emb, projection, start_emb)

    tok_flat = tokens.reshape(ntok).astype(jnp.int32)
    w2s = word2state.astype(jnp.int32)
    zeros = jnp.zeros((c, _LANES), jnp.float32)

    sc_gather = _make_sc_gather(ntok, s, d, c)
    bg, cg, ag, memp = sc_gather(tok_flat, w2s, bp, cp, ap, zeros)

    out, mem = _logmvv(bg, cg, ag, memp)
    return out.reshape(n, t, d, d), mem


# speedup vs baseline: 2.0445x; 1.1752x over previous
import functools

import jax
import jax.numpy as jnp
from jax import lax
from jax.experimental import pallas as pl
from jax.experimental.pallas import tpu as pltpu
from jax.experimental.pallas import tpu_sc as plsc

_NC = 2
_NS = 16
_NW = _NC * _NS
_LANES = 16


def _log_softmax(x):
    m = jnp.max(x, axis=-1, keepdims=True)
    s = jnp.sum(jnp.exp(x - m), axis=-1, keepdims=True)
    return x - m - jnp.log(s)


def _precompute_body(se_ref, ne_ref, proj_ref, start_ref, tbx_ref, ap_ref):
    se = se_ref[...]
    ne = ne_ref[...]
    proj = proj_ref[...]
    start = start_ref[...]
    bp = _log_softmax(jnp.dot(se, proj, preferred_element_type=jnp.float32))
    cp = _log_softmax(jnp.dot(ne, proj, preferred_element_type=jnp.float32))
    tbx_ref[...] = jnp.concatenate([bp, cp], axis=-1)
    ap = jnp.dot(se, start, preferred_element_type=jnp.float32)
    ap_ref[...] = jnp.broadcast_to(ap, (ap.shape[0], _LANES))


def _precompute(state_emb, next_state_emb, projection, start_emb):
    c, h = state_emb.shape
    d = projection.shape[1]
    blk = 2048
    grid = c // blk
    return pl.pallas_call(
        _precompute_body,
        grid=(grid,),
        in_specs=[
            pl.BlockSpec((blk, h), lambda i: (i, 0)),
            pl.BlockSpec((blk, h), lambda i: (i, 0)),
            pl.BlockSpec((h, d), lambda i: (0, 0)),
            pl.BlockSpec((h, 1), lambda i: (0, 0)),
        ],
        out_specs=[
            pl.BlockSpec((blk, 2 * d), lambda i: (i, 0)),
            pl.BlockSpec((blk, _LANES), lambda i: (i, 0)),
        ],
        out_shape=[
            jax.ShapeDtypeStruct((c, 2 * d), jnp.float32),
            jax.ShapeDtypeStruct((c, _LANES), jnp.float32),
        ],
    )(state_emb, next_state_emb, projection, start_emb.reshape(h, 1))


def _make_sc_gather(ntok, s, d, c):
    tpw = ntok // _NW
    rows_pc = c // _NS
    mesh = plsc.VectorSubcoreMesh(core_axis_name="c", subcore_axis_name="s")

    @functools.partial(
        pl.kernel,
        out_type=(
            jax.ShapeDtypeStruct((ntok * s, 2 * d), jnp.float32),
            jax.ShapeDtypeStruct((ntok, s, _LANES), jnp.float32),
            jax.ShapeDtypeStruct((_NC, c, _LANES), jnp.float32),
        ),
        mesh=mesh,
        scratch_types=[
            pltpu.VMEM((tpw,), jnp.int32),
            pltpu.VMEM((tpw, s), jnp.int32),
            pltpu.VMEM((tpw * s, 2 * d), jnp.float32),
            pltpu.VMEM((tpw, s, _LANES), jnp.float32),
            pltpu.VMEM((tpw, s, _LANES), jnp.float32),
            pltpu.VMEM((rows_pc, _LANES), jnp.float32),
            pltpu.VMEM_SHARED((c, _LANES), jnp.float32),
            pltpu.SemaphoreType.DMA,
            pltpu.SemaphoreType.DMA,
        ],
        compiler_params=pltpu.CompilerParams(use_tc_tiling_on_sc=False),
    )
    def sc_gather(tok_hbm, w2s_hbm, tbx_hbm, ap_hbm,
                  tbg_out, ag_out, mem_out,
                  tok_v, st_v, tbg_v, ag_v, ea_v, z_v, mem_sh,
                  sem_b, sem_a):
        cid = lax.axis_index("c")
        sid = lax.axis_index("s")
        wid = sid * _NC + cid
        base = wid * tpw

        pltpu.sync_copy(tok_hbm.at[pl.ds(base, tpw)], tok_v)
        pltpu.async_copy(w2s_hbm.at[tok_v], st_v, sem_b).wait()

        descs = []
        for i in range(tpw):
            descs.append(pltpu.async_copy(tbx_hbm.at[st_v.at[i]],
                                          tbg_v.at[pl.ds(i * s, s)], sem_b))
        ag_descs = []
        for i in range(tpw):
            ag_descs.append(pltpu.async_copy(ap_hbm.at[st_v.at[i]],
                                             ag_v.at[i], sem_a))

        zv = jnp.zeros((_LANES,), jnp.float32)

        def zbody(j, acc):
            z_v[j] = zv
            return acc

        lax.fori_loop(0, rows_pc, zbody, jnp.int32(0))
        pltpu.sync_copy(z_v, mem_sh.at[pl.ds(sid * rows_pc, rows_pc)])

        for de in ag_descs:
            de.wait()

        neg_inf = jnp.full((_LANES,), -jnp.inf, jnp.float32)
        for i in range(tpw):
            m = lax.fori_loop(
                0, s, lambda j, acc: jnp.maximum(acc, ag_v[i, j]), neg_inf)

            def zsum(j, acc, i=i, m=m):
                eu = jnp.exp(ag_v[i, j] - m)
                ea_v[i, j] = eu
                return acc + eu

            z = lax.fori_loop(0, s, zsum, jnp.zeros((_LANES,), jnp.float32))
            rz = jnp.float32(1.0) / z

            def sbody(j, acc, i=i, rz=rz):
                ea_v[i, j] = ea_v[i, j] * rz
                return acc

            lax.fori_loop(0, s, sbody, jnp.int32(0))

        for de in descs:
            de.wait()

        pltpu.sync_copy(tbg_v, tbg_out.at[pl.ds(base * s, tpw * s)])
        pltpu.sync_copy(ag_v, ag_out.at[pl.ds(base, tpw)])

        plsc.subcore_barrier()
        for i in range(tpw):
            pltpu.sync_copy(ea_v.at[i], mem_sh.at[st_v.at[i]], add=True)
        plsc.subcore_barrier()

        @pl.when((sid == 0) & (cid == 0))
        def _():
            pltpu.sync_copy(mem_sh, mem_out.at[0])

        @pl.when((sid == 0) & (cid == 1))
        def _():
            pltpu.sync_copy(mem_sh, mem_out.at[1])

    return sc_gather


def _logmvv_body(tbg_ref, ag_ref, memp_ref, out_ref, mem_ref):
    tbs, d2 = tbg_ref.shape
    d = d2 // 2
    _, s, _ = ag_ref.shape
    tb = tbs // s
    tbg = tbg_ref[...].reshape(tb, s, d2)
    bg = tbg[:, :, :d]
    cg = tbg[:, :, d:]
    ag = ag_ref[:, :, 0]
    a = _log_softmax(ag)
    m_a = jnp.max(a, axis=-1, keepdims=True)
    ea = jnp.exp(a - m_a)
    m_b = jnp.max(bg, axis=1, keepdims=True)
    m_c = jnp.max(cg, axis=1, keepdims=True)
    eb = jnp.exp(bg - m_b)
    ec = jnp.exp(cg - m_c)
    w = ea[:, :, None] * eb
    sm = lax.dot_general(w, ec,
                         dimension_numbers=(((1,), (1,)), ((0,), (0,))),
                         preferred_element_type=jnp.float32)
    out_ref[...] = (m_a[:, :, None] + m_b.reshape(tb, d, 1)
                    + m_c.reshape(tb, 1, d) + jnp.log(sm + 1e-30))
    mem_ref[...] = (memp_ref[0, :, 0] + memp_ref[1, :, 0])[None, None, :]


def _logmvv(tbg, ag, memp, s, d):
    ntok = ag.shape[0]
    c = memp.shape[1]
    tb = 8
    grid = ntok // tb
    cblk = c // grid
    out, mem = pl.pallas_call(
        _logmvv_body,
        grid=(grid,),
        in_specs=[
            pl.BlockSpec((tb * s, 2 * d), lambda i: (i, 0)),
            pl.BlockSpec((tb, s, _LANES), lambda i: (i, 0, 0)),
            pl.BlockSpec((_NC, cblk, _LANES), lambda i: (0, i, 0)),
        ],
        out_specs=[
            pl.BlockSpec((tb, d, d), lambda i: (i, 0, 0)),
            pl.BlockSpec((1, 1, cblk), lambda i: (i, 0, 0)),
        ],
        out_shape=[
            jax.ShapeDtypeStruct((ntok, d, d), jnp.float32),
            jax.ShapeDtypeStruct((grid, 1, cblk), jnp.float32),
        ],
    )(tbg, ag, memp)
    return out, mem.reshape(c)


def kernel(tokens, word2state, state_emb, next_state_emb, projection,
           start_emb):
    n, t = tokens.shape
    c, h = state_emb.shape
    s = word2state.shape[1]
    d = projection.shape[1]
    ntok = n * t

    tbx, ap = _precompute(state_emb, next_state_emb, projection, start_emb)

    tok_flat = tokens.reshape(ntok).astype(jnp.int32)
    w2s = word2state.astype(jnp.int32)

    sc_gather = _make_sc_gather(ntok, s, d, c)
    tbg, ag, memp = sc_gather(tok_flat, w2s, tbx, ap)

    out, mem = _logmvv(tbg, ag, memp, s, d)
    return out.reshape(n, t, d, d), mem


# speedup vs baseline: 2.2518x; 1.1014x over previous
import functools

import jax
import jax.numpy as jnp
from jax import lax
from jax.experimental import pallas as pl
from jax.experimental.pallas import tpu as pltpu
from jax.experimental.pallas import tpu_sc as plsc

_NC = 2
_NS = 16
_NW = _NC * _NS
_LANES = 16


def _log_softmax(x):
    m = jnp.max(x, axis=-1, keepdims=True)
    s = jnp.sum(jnp.exp(x - m), axis=-1, keepdims=True)
    return x - m - jnp.log(s)


def _precompute_body(se_ref, ne_ref, proj_ref, start_ref, tbx_ref, ap_ref):
    se = se_ref[...]
    ne = ne_ref[...]
    proj = proj_ref[...]
    start = start_ref[...]
    bp = _log_softmax(jnp.dot(se, proj, preferred_element_type=jnp.float32))
    cp = _log_softmax(jnp.dot(ne, proj, preferred_element_type=jnp.float32))
    tbx_ref[...] = jnp.concatenate([bp, cp], axis=-1)
    ap = jnp.dot(se, start, preferred_element_type=jnp.float32)
    ap_ref[...] = jnp.broadcast_to(ap, (ap.shape[0], _LANES))


def _precompute(state_emb, next_state_emb, projection, start_emb):
    c, h = state_emb.shape
    d = projection.shape[1]
    blk = 2048
    grid = c // blk
    return pl.pallas_call(
        _precompute_body,
        grid=(grid,),
        in_specs=[
            pl.BlockSpec((blk, h), lambda i: (i, 0)),
            pl.BlockSpec((blk, h), lambda i: (i, 0)),
            pl.BlockSpec((h, d), lambda i: (0, 0)),
            pl.BlockSpec((h, 1), lambda i: (0, 0)),
        ],
        out_specs=[
            pl.BlockSpec((blk, 2 * d), lambda i: (i, 0)),
            pl.BlockSpec((blk, _LANES), lambda i: (i, 0)),
        ],
        out_shape=[
            jax.ShapeDtypeStruct((c, 2 * d), jnp.float32),
            jax.ShapeDtypeStruct((c, _LANES), jnp.float32),
        ],
    )(state_emb, next_state_emb, projection, start_emb.reshape(h, 1))


def _make_sc_gather(ntok, s, d, c):
    tpw = ntok // _NW
    rows_pc = c // _NS
    mesh = plsc.VectorSubcoreMesh(core_axis_name="c", subcore_axis_name="s")

    @functools.partial(
        pl.kernel,
        out_type=(
            jax.ShapeDtypeStruct((ntok * s, 2 * d), jnp.float32),
            jax.ShapeDtypeStruct((ntok, s, _LANES), jnp.float32),
            jax.ShapeDtypeStruct((_NC, c, _LANES), jnp.float32),
        ),
        mesh=mesh,
        scratch_types=[
            pltpu.VMEM((tpw,), jnp.int32),
            pltpu.VMEM((tpw, s), jnp.int32),
            pltpu.VMEM((tpw * s, 2 * d), jnp.float32),
            pltpu.VMEM((tpw, s, _LANES), jnp.float32),
            pltpu.VMEM((tpw, s, _LANES), jnp.float32),
            pltpu.VMEM((rows_pc, _LANES), jnp.float32),
            pltpu.VMEM_SHARED((c, _LANES), jnp.float32),
            pltpu.SemaphoreType.DMA,
            pltpu.SemaphoreType.DMA,
        ],
        compiler_params=pltpu.CompilerParams(use_tc_tiling_on_sc=False),
    )
    def sc_gather(tok_hbm, w2s_hbm, tbx_hbm, ap_hbm,
                  tbg_out, ag_out, mem_out,
                  tok_v, st_v, tbg_v, ag_v, ea_v, z_v, mem_sh,
                  sem_b, sem_a):
        cid = lax.axis_index("c")
        sid = lax.axis_index("s")
        wid = sid * _NC + cid
        base = wid * tpw

        pltpu.sync_copy(tok_hbm.at[pl.ds(base, tpw)], tok_v)
        pltpu.async_copy(w2s_hbm.at[tok_v], st_v, sem_b).wait()

        descs = []
        for i in range(tpw):
            descs.append(pltpu.async_copy(tbx_hbm.at[st_v.at[i]],
                                          tbg_v.at[pl.ds(i * s, s)], sem_b))
        ag_descs = []
        for i in range(tpw):
            ag_descs.append(pltpu.async_copy(ap_hbm.at[st_v.at[i]],
                                             ag_v.at[i], sem_a))

        zv = jnp.zeros((_LANES,), jnp.float32)

        def zbody(j, acc):
            z_v[j] = zv
            return acc

        lax.fori_loop(0, rows_pc, zbody, jnp.int32(0))
        pltpu.sync_copy(z_v, mem_sh.at[pl.ds(sid * rows_pc, rows_pc)])

        for de in ag_descs:
            de.wait()

        neg_inf = jnp.full((_LANES,), -jnp.inf, jnp.float32)
        for i in range(tpw):
            m = lax.fori_loop(
                0, s, lambda j, acc: jnp.maximum(acc, ag_v[i, j]), neg_inf)

            def zsum(j, acc, i=i, m=m):
                eu = jnp.exp(ag_v[i, j] - m)
                ea_v[i, j] = eu
                return acc + eu

            z = lax.fori_loop(0, s, zsum, jnp.zeros((_LANES,), jnp.float32))
            rz = jnp.float32(1.0) / z

            def sbody(j, acc, i=i, rz=rz):
                ea_v[i, j] = ea_v[i, j] * rz
                return acc

            lax.fori_loop(0, s, sbody, jnp.int32(0))

        for de in descs:
            de.wait()

        pltpu.sync_copy(tbg_v, tbg_out.at[pl.ds(base * s, tpw * s)])
        pltpu.sync_copy(ag_v, ag_out.at[pl.ds(base, tpw)])

        plsc.subcore_barrier()
        for i in range(tpw):
            pltpu.sync_copy(ea_v.at[i], mem_sh.at[st_v.at[i]], add=True)
        plsc.subcore_barrier()

        @pl.when((sid == 0) & (cid == 0))
        def _():
            pltpu.sync_copy(mem_sh, mem_out.at[0])

        @pl.when((sid == 0) & (cid == 1))
        def _():
            pltpu.sync_copy(mem_sh, mem_out.at[1])

    return sc_gather


def _logmvv_body(tbg_ref, ag_ref, memp_ref, out_ref, mem_ref):
    tbs, d2 = tbg_ref.shape
    d = d2 // 2
    _, s, _ = ag_ref.shape
    tb = tbs // s
    tbg = tbg_ref[...].reshape(tb, s, d2)
    ag = ag_ref[:, :, 0]
    a = _log_softmax(ag)
    lane = lax.broadcasted_iota(jnp.int32, (tb, s, d2), 2)
    a3 = jnp.broadcast_to(a[:, :, None], (tb, s, d2))
    abg = tbg + jnp.where(lane < d, a3, jnp.float32(0.0))
    m = jnp.max(abg, axis=1, keepdims=True)
    e = jnp.exp(abg - m)
    w = e[:, :, :d]
    ec = e[:, :, d:]
    sm = lax.dot_general(w, ec,
                         dimension_numbers=(((1,), (1,)), ((0,), (0,))),
                         preferred_element_type=jnp.float32)
    out_ref[...] = (m[:, 0, :d].reshape(tb, d, 1)
                    + m[:, 0, d:].reshape(tb, 1, d) + jnp.log(sm + 1e-30))
    mem_ref[...] = (memp_ref[0, :, 0] + memp_ref[1, :, 0])[None, None, :]


def _logmvv(tbg, ag, memp, s, d):
    ntok = ag.shape[0]
    c = memp.shape[1]
    tb = 16
    grid = ntok // tb
    cblk = c // grid
    out, mem = pl.pallas_call(
        _logmvv_body,
        grid=(grid,),
        in_specs=[
            pl.BlockSpec((tb * s, 2 * d), lambda i: (i, 0)),
            pl.BlockSpec((tb, s, _LANES), lambda i: (i, 0, 0)),
            pl.BlockSpec((_NC, cblk, _LANES), lambda i: (0, i, 0)),
        ],
        out_specs=[
            pl.BlockSpec((tb, d, d), lambda i: (i, 0, 0)),
            pl.BlockSpec((1, 1, cblk), lambda i: (i, 0, 0)),
        ],
        out_shape=[
            jax.ShapeDtypeStruct((ntok, d, d), jnp.float32),
            jax.ShapeDtypeStruct((grid, 1, cblk), jnp.float32),
        ],
    )(tbg, ag, memp)
    return out, mem.reshape(c)


def kernel(tokens, word2state, state_emb, next_state_emb, projection,
           start_emb):
    n, t = tokens.shape
    c, h = state_emb.shape
    s = word2state.shape[1]
    d = projection.shape[1]
    ntok = n * t

    tbx, ap = _precompute(state_emb, next_state_emb, projection, start_emb)

    tok_flat = tokens.reshape(ntok).astype(jnp.int32)
    w2s = word2state.astype(jnp.int32)

    sc_gather = _make_sc_gather(ntok, s, d, c)
    tbg, ag, memp = sc_gather(tok_flat, w2s, tbx, ap)

    out, mem = _logmvv(tbg, ag, memp, s, d)
    return out.reshape(n, t, d, d), mem


# speedup vs baseline: 2.5480x; 1.1316x over previous
import functools

import jax
import jax.numpy as jnp
from jax import lax
from jax.experimental import pallas as pl
from jax.experimental.pallas import tpu as pltpu
from jax.experimental.pallas import tpu_sc as plsc

_NC = 2
_NS = 16
_NW = _NC * _NS
_LANES = 16


def _log_softmax(x):
    m = jnp.max(x, axis=-1, keepdims=True)
    s = jnp.sum(jnp.exp(x - m), axis=-1, keepdims=True)
    return x - m - jnp.log(s)


def _precompute_body(se_ref, ne_ref, proj_ref, start_ref, tbx_ref, ap_ref):
    se = se_ref[...]
    ne = ne_ref[...]
    proj = proj_ref[...]
    start = start_ref[...]
    bp = _log_softmax(jnp.dot(se, proj, preferred_element_type=jnp.float32))
    cp = _log_softmax(jnp.dot(ne, proj, preferred_element_type=jnp.float32))
    tbx_ref[...] = jnp.concatenate([bp, cp], axis=-1)
    ap = jnp.dot(se, start, preferred_element_type=jnp.float32)
    ap_ref[...] = jnp.broadcast_to(ap, (ap.shape[0], _LANES))


def _precompute(state_emb, next_state_emb, projection, start_emb):
    c, h = state_emb.shape
    d = projection.shape[1]
    blk = 2048
    grid = c // blk
    return pl.pallas_call(
        _precompute_body,
        grid=(grid,),
        in_specs=[
            pl.BlockSpec((blk, h), lambda i: (i, 0)),
            pl.BlockSpec((blk, h), lambda i: (i, 0)),
            pl.BlockSpec((h, d), lambda i: (0, 0)),
            pl.BlockSpec((h, 1), lambda i: (0, 0)),
        ],
        out_specs=[
            pl.BlockSpec((blk, 2 * d), lambda i: (i, 0)),
            pl.BlockSpec((blk, _LANES), lambda i: (i, 0)),
        ],
        out_shape=[
            jax.ShapeDtypeStruct((c, 2 * d), jnp.float32),
            jax.ShapeDtypeStruct((c, _LANES), jnp.float32),
        ],
    )(state_emb, next_state_emb, projection, start_emb.reshape(h, 1))


def _make_sc_gather(ntok, s, d, c):
    tpw = ntok // _NW
    rows_pc = c // _NS
    mesh = plsc.VectorSubcoreMesh(core_axis_name="c", subcore_axis_name="s")

    @functools.partial(
        pl.kernel,
        out_type=(
            jax.ShapeDtypeStruct((ntok * s, 2 * d), jnp.float32),
            jax.ShapeDtypeStruct((ntok * s * _LANES // 128, 128),
                                 jnp.float32),
            jax.ShapeDtypeStruct((_NC, c * _LANES // 128, 128),
                                 jnp.float32),
        ),
        mesh=mesh,
        scratch_types=[
            pltpu.VMEM((tpw,), jnp.int32),
            pltpu.VMEM((tpw, s), jnp.int32),
            pltpu.VMEM((tpw * s, 2 * d), jnp.float32),
            pltpu.VMEM((tpw, s, _LANES), jnp.float32),
            pltpu.VMEM((tpw * s, _LANES), jnp.float32),
            pltpu.VMEM((tpw * s * _LANES // 128, 128), jnp.float32),
            pltpu.VMEM((64, _LANES), jnp.float32),
            pltpu.VMEM((rows_pc * _LANES // 128, 128), jnp.float32),
            pltpu.VMEM_SHARED((c, _LANES), jnp.float32),
            pltpu.SemaphoreType.DMA,
            pltpu.SemaphoreType.DMA,
        ],
        compiler_params=pltpu.CompilerParams(use_tc_tiling_on_sc=False),
    )
    def sc_gather(tok_hbm, w2s_hbm, tbx_hbm, ap_hbm,
                  tbg_out, eac_out, mem_out,
                  tok_v, st_v, tbg_v, ag_v, ea_v, eac_v, z_v, z3_v, mem_sh,
                  sem_b, sem_a):
        cid = lax.axis_index("c")
        sid = lax.axis_index("s")
        wid = sid * _NC + cid
        base = wid * tpw

        pltpu.sync_copy(tok_hbm.at[pl.ds(base, tpw)], tok_v)
        pltpu.async_copy(w2s_hbm.at[tok_v], st_v, sem_b).wait()

        descs = []
        for i in range(tpw):
            descs.append(pltpu.async_copy(tbx_hbm.at[st_v.at[i]],
                                          tbg_v.at[pl.ds(i * s, s)], sem_b))
        ag_descs = []
        for i in range(tpw):
            ag_descs.append(pltpu.async_copy(ap_hbm.at[st_v.at[i]],
                                             ag_v.at[i], sem_a))

        zv = jnp.zeros((_LANES,), jnp.float32)

        def zbody(j, acc):
            z_v[j] = zv
            return acc

        lax.fori_loop(0, 64, zbody, jnp.int32(0))

        def zcopy(kk, acc):
            pltpu.sync_copy(
                z_v, mem_sh.at[pl.ds(sid * rows_pc + kk * 64, 64)])
            return acc

        lax.fori_loop(0, rows_pc // 64, zcopy, jnp.int32(0))

        for de in ag_descs:
            de.wait()

        neg_inf = jnp.full((_LANES,), -jnp.inf, jnp.float32)
        for i in range(tpw):
            m = lax.fori_loop(
                0, s, lambda j, acc: jnp.maximum(acc, ag_v[i, j]), neg_inf)

            def zsum(j, acc, i=i, m=m):
                eu = jnp.exp(ag_v[i, j] - m)
                ea_v[i * s + j] = eu
                return acc + eu

            z = lax.fori_loop(0, s, zsum, jnp.zeros((_LANES,), jnp.float32))
            rz = jnp.float32(1.0) / z

            def sbody(j, acc, i=i, rz=rz):
                ea_v[i * s + j] = ea_v[i * s + j] * rz
                return acc

            lax.fori_loop(0, s, sbody, jnp.int32(0))

        def ebody(q, acc):
            for k in range(128 // _LANES):
                eac_v[q, pl.ds(k * _LANES, _LANES)] = ea_v[q * 8 + k]
            return acc

        lax.fori_loop(0, tpw * s * _LANES // 128, ebody, jnp.int32(0))

        for de in descs:
            de.wait()

        pltpu.sync_copy(tbg_v, tbg_out.at[pl.ds(base * s, tpw * s)])
        erows = tpw * s * _LANES // 128
        pltpu.sync_copy(eac_v, eac_out.at[pl.ds(wid * erows, erows)])

        plsc.subcore_barrier()
        for i in range(tpw):
            pltpu.sync_copy(ea_v.at[pl.ds(i * s, s)],
                            mem_sh.at[st_v.at[i]], add=True)
        plsc.subcore_barrier()

        rows_pt = rows_pc * _LANES // 128

        def dchunk(kk, acc):
            pltpu.sync_copy(
                mem_sh.at[pl.ds(sid * rows_pc + kk * 64, 64)], z_v)

            def rbody(q, acc2, kk=kk):
                for k in range(128 // _LANES):
                    z3_v[kk * 8 + q, pl.ds(k * _LANES, _LANES)] = \
                        z_v[q * 8 + k]
                return acc2

            lax.fori_loop(0, 8, rbody, jnp.int32(0))
            return acc

        lax.fori_loop(0, rows_pc // 64, dchunk, jnp.int32(0))

        @pl.when(cid == 0)
        def _():
            pltpu.sync_copy(z3_v, mem_out.at[0].at[pl.ds(sid * rows_pt,
                                                         rows_pt)])

        @pl.when(cid == 1)
        def _():
            pltpu.sync_copy(z3_v, mem_out.at[1].at[pl.ds(sid * rows_pt,
                                                         rows_pt)])

    return sc_gather


def _logmvv_body(s, tbg_ref, eac_ref, memp_ref, out_ref, mem_ref):
    tbs, d2 = tbg_ref.shape
    d = d2 // 2
    tb = tbs // s
    tbg = tbg_ref[...].reshape(tb, s, d2)
    ez = eac_ref[...]
    ea = ez.reshape(ez.shape[0], 128 // _LANES,
                    _LANES)[:, :, 0].reshape(tb, s)
    a = jnp.log(ea)
    lane = lax.broadcasted_iota(jnp.int32, (tb, s, d2), 2)
    a3 = jnp.broadcast_to(a[:, :, None], (tb, s, d2))
    abg = tbg + jnp.where(lane < d, a3, jnp.float32(0.0))
    m = jnp.max(abg, axis=1, keepdims=True)
    e = jnp.exp(abg - m)
    w = e[:, :, :d]
    ec = e[:, :, d:]
    sm = lax.dot_general(w, ec,
                         dimension_numbers=(((1,), (1,)), ((0,), (0,))),
                         preferred_element_type=jnp.float32)
    out_ref[...] = (m[:, 0, :d].reshape(tb, d, 1)
                    + m[:, 0, d:].reshape(tb, 1, d) + jnp.log(sm + 1e-30))
    mp = memp_ref[0] + memp_ref[1]
    r = mp.shape[0]
    lane0 = mp.reshape(r, 128 // _LANES, _LANES)[:, :, 0]
    mem_ref[...] = lane0.reshape(1, 1, r * (128 // _LANES))


def _logmvv(tbg, eac, memp, ntok, s, d, c):
    tb = 16
    grid = ntok // tb
    cblk = c // grid
    rblk = cblk * _LANES // 128
    out, mem = pl.pallas_call(
        functools.partial(_logmvv_body, s),
        grid=(grid,),
        in_specs=[
            pl.BlockSpec((tb * s, 2 * d), lambda i: (i, 0)),
            pl.BlockSpec((tb * s * _LANES // 128, 128), lambda i: (i, 0)),
            pl.BlockSpec((_NC, rblk, 128), lambda i: (0, i, 0)),
        ],
        out_specs=[
            pl.BlockSpec((tb, d, d), lambda i: (i, 0, 0)),
            pl.BlockSpec((1, 1, cblk), lambda i: (i, 0, 0)),
        ],
        out_shape=[
            jax.ShapeDtypeStruct((ntok, d, d), jnp.float32),
            jax.ShapeDtypeStruct((grid, 1, cblk), jnp.float32),
        ],
    )(tbg, eac, memp)
    return out, mem.reshape(c)


def kernel(tokens, word2state, state_emb, next_state_emb, projection,
           start_emb):
    n, t = tokens.shape
    c, h = state_emb.shape
    s = word2state.shape[1]
    d = projection.shape[1]
    ntok = n * t

    tbx, ap = _precompute(state_emb, next_state_emb, projection, start_emb)

    tok_flat = tokens.reshape(ntok).astype(jnp.int32)
    w2s = word2state.astype(jnp.int32)

    sc_gather = _make_sc_gather(ntok, s, d, c)
    tbg, eac, memp = sc_gather(tok_flat, w2s, tbx, ap)

    out, mem = _logmvv(tbg, eac, memp, ntok, s, d, c)
    return out.reshape(n, t, d, d), mem


# speedup vs baseline: 2.6308x; 1.0325x over previous
import functools

import jax
import jax.numpy as jnp
from jax import lax
from jax.experimental import pallas as pl
from jax.experimental.pallas import tpu as pltpu
from jax.experimental.pallas import tpu_sc as plsc

_NC = 2
_NS = 16
_NW = _NC * _NS
_LANES = 16


def _log_softmax(x):
    m = jnp.max(x, axis=-1, keepdims=True)
    s = jnp.sum(jnp.exp(x - m), axis=-1, keepdims=True)
    return x - m - jnp.log(s)


def _precompute_body(se_ref, ne_ref, proj_ref, start_ref, tbx_ref, ap_ref):
    se = se_ref[...]
    ne = ne_ref[...]
    proj = proj_ref[...]
    start = start_ref[...]
    bp = _log_softmax(jnp.dot(se, proj, preferred_element_type=jnp.float32))
    cp = _log_softmax(jnp.dot(ne, proj, preferred_element_type=jnp.float32))
    tbx_ref[...] = jnp.concatenate([bp, cp], axis=-1)
    ap = jnp.dot(se, start, preferred_element_type=jnp.float32)
    ap_ref[...] = jnp.broadcast_to(ap, (ap.shape[0], _LANES))


def _precompute(state_emb, next_state_emb, projection, start_emb):
    c, h = state_emb.shape
    d = projection.shape[1]
    blk = 2048
    grid = c // blk
    return pl.pallas_call(
        _precompute_body,
        grid=(grid,),
        in_specs=[
            pl.BlockSpec((blk, h), lambda i: (i, 0)),
            pl.BlockSpec((blk, h), lambda i: (i, 0)),
            pl.BlockSpec((h, d), lambda i: (0, 0)),
            pl.BlockSpec((h, 1), lambda i: (0, 0)),
        ],
        out_specs=[
            pl.BlockSpec((blk, 2 * d), lambda i: (i, 0)),
            pl.BlockSpec((blk, _LANES), lambda i: (i, 0)),
        ],
        out_shape=[
            jax.ShapeDtypeStruct((c, 2 * d), jnp.float32),
            jax.ShapeDtypeStruct((c, _LANES), jnp.float32),
        ],
    )(state_emb, next_state_emb, projection, start_emb.reshape(h, 1))


def _make_sc_gather(ntok, s, d, c):
    tpw = ntok // _NW
    rows_pc = c // _NS
    mesh = plsc.VectorSubcoreMesh(core_axis_name="c", subcore_axis_name="s")

    @functools.partial(
        pl.kernel,
        out_type=(
            jax.ShapeDtypeStruct((ntok, s, 2 * d), jnp.float32),
            jax.ShapeDtypeStruct((ntok * s * _LANES // 128, 128),
                                 jnp.float32),
            jax.ShapeDtypeStruct((_NC, c * _LANES // 128, 128),
                                 jnp.float32),
        ),
        mesh=mesh,
        scratch_types=[
            pltpu.VMEM((tpw,), jnp.int32),
            pltpu.VMEM((tpw, s), jnp.int32),
            pltpu.VMEM((tpw, s, 2 * d), jnp.float32),
            pltpu.VMEM((tpw, s, _LANES), jnp.float32),
            pltpu.VMEM((tpw * s, _LANES), jnp.float32),
            pltpu.VMEM((tpw * s * _LANES // 128, 128), jnp.float32),
            pltpu.VMEM((64, _LANES), jnp.float32),
            pltpu.VMEM((rows_pc * _LANES // 128, 128), jnp.float32),
            pltpu.VMEM_SHARED((c, _LANES), jnp.float32),
            pltpu.SemaphoreType.DMA,
            pltpu.SemaphoreType.DMA,
        ],
        compiler_params=pltpu.CompilerParams(use_tc_tiling_on_sc=False),
    )
    def sc_gather(tok_hbm, w2s_hbm, tbx_hbm, ap_hbm,
                  tbg_out, eac_out, mem_out,
                  tok_v, st_v, tbg_v, ag_v, ea_v, eac_v, z_v, z3_v, mem_sh,
                  sem_b, sem_a):
        cid = lax.axis_index("c")
        sid = lax.axis_index("s")
        wid = sid * _NC + cid
        base = wid * tpw

        pltpu.sync_copy(tok_hbm.at[pl.ds(base, tpw)], tok_v)
        pltpu.async_copy(w2s_hbm.at[tok_v], st_v, sem_b).wait()

        descs = []
        ag_descs = []
        for i in range(tpw):
            descs.append(pltpu.async_copy(tbx_hbm.at[st_v.at[i]],
                                          tbg_v.at[i], sem_b))
        for i in range(tpw):
            ag_descs.append(pltpu.async_copy(ap_hbm.at[st_v.at[i]],
                                             ag_v.at[i], sem_a))

        zv = jnp.zeros((_LANES,), jnp.float32)

        def zbody(j, acc):
            z_v[j] = zv
            return acc

        lax.fori_loop(0, 64, zbody, jnp.int32(0))

        def zcopy(kk, acc):
            pltpu.sync_copy(
                z_v, mem_sh.at[pl.ds(sid * rows_pc + kk * 64, 64)])
            return acc

        lax.fori_loop(0, rows_pc // 64, zcopy, jnp.int32(0))

        for de in ag_descs:
            de.wait()

        neg_inf = jnp.full((_LANES,), -jnp.inf, jnp.float32)
        unr = 128 // _LANES
        for i in range(tpw):
            def mbody(q, acc, i=i):
                for k in range(unr):
                    acc = jnp.maximum(acc, ag_v[i, q * unr + k])
                return acc

            m = lax.fori_loop(0, s // unr, mbody, neg_inf)

            def zsum(q, acc, i=i, m=m):
                for k in range(unr):
                    eu = jnp.exp(ag_v[i, q * unr + k] - m)
                    ea_v[i * s + q * unr + k] = eu
                    acc = acc + eu
                return acc

            z = lax.fori_loop(0, s // unr, zsum,
                              jnp.zeros((_LANES,), jnp.float32))
            rz = jnp.float32(1.0) / z

            def spack(q, acc, i=i, rz=rz):
                for k in range(unr):
                    v = ea_v[i * s + q * unr + k] * rz
                    ea_v[i * s + q * unr + k] = v
                    eac_v[i * (s // unr) + q, pl.ds(k * _LANES, _LANES)] = v
                return acc

            lax.fori_loop(0, s // unr, spack, jnp.int32(0))

        for de in descs:
            de.wait()

        pltpu.sync_copy(tbg_v, tbg_out.at[pl.ds(base, tpw)])
        erows = tpw * s * _LANES // 128
        pltpu.sync_copy(eac_v, eac_out.at[pl.ds(wid * erows, erows)])

        plsc.subcore_barrier()
        for i in range(tpw):
            pltpu.sync_copy(ea_v.at[pl.ds(i * s, s)],
                            mem_sh.at[st_v.at[i]], add=True)
        plsc.subcore_barrier()

        rows_pt = rows_pc * _LANES // 128

        def dchunk(kk, acc):
            pltpu.sync_copy(
                mem_sh.at[pl.ds(sid * rows_pc + kk * 64, 64)], z_v)

            def rbody(q, acc2, kk=kk):
                for k in range(128 // _LANES):
                    z3_v[kk * 8 + q, pl.ds(k * _LANES, _LANES)] = \
                        z_v[q * 8 + k]
                return acc2

            lax.fori_loop(0, 8, rbody, jnp.int32(0))
            return acc

        lax.fori_loop(0, rows_pc // 64, dchunk, jnp.int32(0))

        @pl.when(cid == 0)
        def _():
            pltpu.sync_copy(z3_v, mem_out.at[0].at[pl.ds(sid * rows_pt,
                                                         rows_pt)])

        @pl.when(cid == 1)
        def _():
            pltpu.sync_copy(z3_v, mem_out.at[1].at[pl.ds(sid * rows_pt,
                                                         rows_pt)])

    return sc_gather


def _logmvv_body(s, tbg_ref, eac_ref, memp_ref, out_ref, mem_ref):
    tb, _, d2 = tbg_ref.shape
    d = d2 // 2
    tbg = tbg_ref[...]
    ez = eac_ref[...]
    ea = ez.reshape(ez.shape[0], 128 // _LANES,
                    _LANES)[:, :, 0].reshape(tb, s)
    a = jnp.log(ea)
    lane = lax.broadcasted_iota(jnp.int32, (tb, s, d2), 2)
    a3 = jnp.broadcast_to(a[:, :, None], (tb, s, d2))
    abg = tbg + jnp.where(lane < d, a3, jnp.float32(0.0))
    m = jnp.max(abg, axis=1, keepdims=True)
    e = jnp.exp(abg - m)
    w = e[:, :, :d]
    ec = e[:, :, d:]
    sm = lax.dot_general(w, ec,
                         dimension_numbers=(((1,), (1,)), ((0,), (0,))),
                         preferred_element_type=jnp.float32)
    out_ref[...] = (m[:, 0, :d].reshape(tb, d, 1)
                    + m[:, 0, d:].reshape(tb, 1, d) + jnp.log(sm + 1e-30))
    mp = memp_ref[0] + memp_ref[1]
    r = mp.shape[0]
    lane0 = mp.reshape(r, 128 // _LANES, _LANES)[:, :, 0]
    mem_ref[...] = lane0.reshape(1, 1, r * (128 // _LANES))


def _logmvv(tbg, eac, memp, ntok, s, d, c):
    tb = 16
    grid = ntok // tb
    cblk = c // grid
    rblk = cblk * _LANES // 128
    out, mem = pl.pallas_call(
        functools.partial(_logmvv_body, s),
        grid=(grid,),
        in_specs=[
            pl.BlockSpec((tb, s, 2 * d), lambda i: (i, 0, 0)),
            pl.BlockSpec((tb * s * _LANES // 128, 128), lambda i: (i, 0)),
            pl.BlockSpec((_NC, rblk, 128), lambda i: (0, i, 0)),
        ],
        out_specs=[
            pl.BlockSpec((tb, d, d), lambda i: (i, 0, 0)),
            pl.BlockSpec((1, 1, cblk), lambda i: (i, 0, 0)),
        ],
        out_shape=[
            jax.ShapeDtypeStruct((ntok, d, d), jnp.float32),
            jax.ShapeDtypeStruct((grid, 1, cblk), jnp.float32),
        ],
    )(tbg, eac, memp)
    return out, mem.reshape(c)


def kernel(tokens, word2state, state_emb, next_state_emb, projection,
           start_emb):
    n, t = tokens.shape
    c, h = state_emb.shape
    s = word2state.shape[1]
    d = projection.shape[1]
    ntok = n * t

    tbx, ap = _precompute(state_emb, next_state_emb, projection, start_emb)

    tok_flat = tokens.reshape(ntok).astype(jnp.int32)
    w2s = word2state.astype(jnp.int32)

    sc_gather = _make_sc_gather(ntok, s, d, c)
    tbg, eac, memp = sc_gather(tok_flat, w2s, tbx, ap)

    out, mem = _logmvv(tbg, eac, memp, ntok, s, d, c)
    return out.reshape(n, t, d, d), mem


# speedup vs baseline: 2.7266x; 1.0364x over previous
import functools

import jax
import jax.numpy as jnp
from jax import lax
from jax.experimental import pallas as pl
from jax.experimental.pallas import tpu as pltpu
from jax.experimental.pallas import tpu_sc as plsc

_NC = 2
_NS = 16
_NW = _NC * _NS
_LANES = 16


def _log_softmax(x):
    m = jnp.max(x, axis=-1, keepdims=True)
    s = jnp.sum(jnp.exp(x - m), axis=-1, keepdims=True)
    return x - m - jnp.log(s)


def _precompute_body(se_ref, ne_ref, proj_ref, start_ref, tbx_ref, ap_ref):
    se = se_ref[...]
    ne = ne_ref[...]
    proj = proj_ref[...]
    start = start_ref[...]
    bp = _log_softmax(jnp.dot(se, proj, preferred_element_type=jnp.float32))
    cp = _log_softmax(jnp.dot(ne, proj, preferred_element_type=jnp.float32))
    tbx_ref[...] = jnp.concatenate([bp, cp], axis=-1)
    ap = lax.dot_general(se, start,
                         dimension_numbers=(((1,), (1,)), ((), ())),
                         preferred_element_type=jnp.float32)
    ap_ref[...] = jnp.broadcast_to(ap, (ap.shape[0], _LANES))


def _precompute(state_emb, next_state_emb, projection, start_emb):
    c, h = state_emb.shape
    d = projection.shape[1]
    blk = 2048
    grid = c // blk
    return pl.pallas_call(
        _precompute_body,
        grid=(grid,),
        in_specs=[
            pl.BlockSpec((blk, h), lambda i: (i, 0)),
            pl.BlockSpec((blk, h), lambda i: (i, 0)),
            pl.BlockSpec((h, d), lambda i: (0, 0)),
            pl.BlockSpec((1, h), lambda i: (0, 0)),
        ],
        out_specs=[
            pl.BlockSpec((blk, 2 * d), lambda i: (i, 0)),
            pl.BlockSpec((blk, _LANES), lambda i: (i, 0)),
        ],
        out_shape=[
            jax.ShapeDtypeStruct((c, 2 * d), jnp.float32),
            jax.ShapeDtypeStruct((c, _LANES), jnp.float32),
        ],
    )(state_emb, next_state_emb, projection, start_emb.reshape(1, h))


def _make_sc_gather(ntok, s, d, c):
    tpw = ntok // _NW
    rows_pc = c // _NS
    mesh = plsc.VectorSubcoreMesh(core_axis_name="c", subcore_axis_name="s")

    @functools.partial(
        pl.kernel,
        out_type=(
            jax.ShapeDtypeStruct((ntok, s, 2 * d), jnp.float32),
            jax.ShapeDtypeStruct((ntok * s * _LANES // 128, 128),
                                 jnp.float32),
            jax.ShapeDtypeStruct((_NC, c * _LANES // 128, 128),
                                 jnp.float32),
        ),
        mesh=mesh,
        scratch_types=[
            pltpu.VMEM((tpw,), jnp.int32),
            pltpu.VMEM((tpw, s), jnp.int32),
            pltpu.VMEM((tpw, s, 2 * d), jnp.float32),
            pltpu.VMEM((tpw, s, _LANES), jnp.float32),
            pltpu.VMEM((tpw * s, _LANES), jnp.float32),
            pltpu.VMEM((tpw * s * _LANES // 128, 128), jnp.float32),
            pltpu.VMEM((64, _LANES), jnp.float32),
            pltpu.VMEM((rows_pc * _LANES // 128, 128), jnp.float32),
            pltpu.VMEM_SHARED((c, _LANES), jnp.float32),
            pltpu.SemaphoreType.DMA,
            pltpu.SemaphoreType.DMA,
        ],
        compiler_params=pltpu.CompilerParams(use_tc_tiling_on_sc=False),
    )
    def sc_gather(tok_hbm, w2s_hbm, tbx_hbm, ap_hbm,
                  tbg_out, eac_out, mem_out,
                  tok_v, st_v, tbg_v, ag_v, ea_v, eac_v, z_v, z3_v, mem_sh,
                  sem_b, sem_a):
        cid = lax.axis_index("c")
        sid = lax.axis_index("s")
        wid = sid * _NC + cid
        base = wid * tpw

        pltpu.sync_copy(tok_hbm.at[pl.ds(base, tpw)], tok_v)
        pltpu.async_copy(w2s_hbm.at[tok_v], st_v, sem_b).wait()

        descs = []
        ag_descs = []
        for i in range(tpw):
            descs.append(pltpu.async_copy(tbx_hbm.at[st_v.at[i]],
                                          tbg_v.at[i], sem_b))
        for i in range(tpw):
            ag_descs.append(pltpu.async_copy(ap_hbm.at[st_v.at[i]],
                                             ag_v.at[i], sem_a))

        zv = jnp.zeros((_LANES,), jnp.float32)

        def zbody(j, acc):
            z_v[j] = zv
            return acc

        lax.fori_loop(0, 64, zbody, jnp.int32(0))

        def zcopy(kk, acc):
            pltpu.sync_copy(
                z_v, mem_sh.at[pl.ds(sid * rows_pc + kk * 64, 64)])
            return acc

        lax.fori_loop(0, rows_pc // 64, zcopy, jnp.int32(0))

        for de in ag_descs:
            de.wait()

        neg_inf = jnp.full((_LANES,), -jnp.inf, jnp.float32)
        unr = 128 // _LANES
        for i in range(tpw):
            def mbody(q, acc, i=i):
                for k in range(unr):
                    acc = jnp.maximum(acc, ag_v[i, q * unr + k])
                return acc

            m = lax.fori_loop(0, s // unr, mbody, neg_inf)

            def zsum(q, acc, i=i, m=m):
                for k in range(unr):
                    eu = jnp.exp(ag_v[i, q * unr + k] - m)
                    ea_v[i * s + q * unr + k] = eu
                    acc = acc + eu
                return acc

            z = lax.fori_loop(0, s // unr, zsum,
                              jnp.zeros((_LANES,), jnp.float32))
            rz = jnp.float32(1.0) / z

            def spack(q, acc, i=i, rz=rz):
                for k in range(unr):
                    v = ea_v[i * s + q * unr + k] * rz
                    ea_v[i * s + q * unr + k] = v
                    eac_v[i * (s // unr) + q, pl.ds(k * _LANES, _LANES)] = v
                return acc

            lax.fori_loop(0, s // unr, spack, jnp.int32(0))

        for de in descs:
            de.wait()

        pltpu.sync_copy(tbg_v, tbg_out.at[pl.ds(base, tpw)])
        erows = tpw * s * _LANES // 128
        pltpu.sync_copy(eac_v, eac_out.at[pl.ds(wid * erows, erows)])

        plsc.subcore_barrier()
        for i in range(tpw):
            pltpu.sync_copy(ea_v.at[pl.ds(i * s, s)],
                            mem_sh.at[st_v.at[i]], add=True)
        plsc.subcore_barrier()

        rows_pt = rows_pc * _LANES // 128

        def dchunk(kk, acc):
            pltpu.sync_copy(
                mem_sh.at[pl.ds(sid * rows_pc + kk * 64, 64)], z_v)

            def rbody(q, acc2, kk=kk):
                for k in range(128 // _LANES):
                    z3_v[kk * 8 + q, pl.ds(k * _LANES, _LANES)] = \
                        z_v[q * 8 + k]
                return acc2

            lax.fori_loop(0, 8, rbody, jnp.int32(0))
            return acc

        lax.fori_loop(0, rows_pc // 64, dchunk, jnp.int32(0))

        @pl.when(cid == 0)
        def _():
            pltpu.sync_copy(z3_v, mem_out.at[0].at[pl.ds(sid * rows_pt,
                                                         rows_pt)])

        @pl.when(cid == 1)
        def _():
            pltpu.sync_copy(z3_v, mem_out.at[1].at[pl.ds(sid * rows_pt,
                                                         rows_pt)])

    return sc_gather


def _logmvv_body(s, tbg_ref, eac_ref, memp_ref, out_ref, mem_ref):
    tb, _, d2 = tbg_ref.shape
    d = d2 // 2
    tbg = tbg_ref[...]
    ez = eac_ref[...]
    ea = ez.reshape(ez.shape[0], 128 // _LANES,
                    _LANES)[:, :, 0].reshape(tb, s)
    a = jnp.log(ea)
    lane = lax.broadcasted_iota(jnp.int32, (tb, s, d2), 2)
    a3 = jnp.broadcast_to(a[:, :, None], (tb, s, d2))
    abg = tbg + jnp.where(lane < d, a3, jnp.float32(0.0))
    m = jnp.max(abg, axis=1, keepdims=True)
    e = jnp.exp(abg - m)
    w = e[:, :, :d]
    ec = e[:, :, d:]
    sm = lax.dot_general(w, ec,
                         dimension_numbers=(((1,), (1,)), ((0,), (0,))),
                         preferred_element_type=jnp.float32)
    out_ref[...] = (m[:, 0, :d].reshape(tb, d, 1)
                    + m[:, 0, d:].reshape(tb, 1, d) + jnp.log(sm + 1e-30))
    mp = memp_ref[0] + memp_ref[1]
    r = mp.shape[0]
    lane0 = mp.reshape(r, 128 // _LANES, _LANES)[:, :, 0]
    mem_ref[...] = lane0.reshape(1, 1, r * (128 // _LANES))


def _logmvv(tbg, eac, memp, ntok, s, d, c):
    tb = 32
    grid = ntok // tb
    cblk = c // grid
    rblk = cblk * _LANES // 128
    out, mem = pl.pallas_call(
        functools.partial(_logmvv_body, s),
        grid=(grid,),
        in_specs=[
            pl.BlockSpec((tb, s, 2 * d), lambda i: (i, 0, 0)),
            pl.BlockSpec((tb * s * _LANES // 128, 128), lambda i: (i, 0)),
            pl.BlockSpec((_NC, rblk, 128), lambda i: (0, i, 0)),
        ],
        out_specs=[
            pl.BlockSpec((tb, d, d), lambda i: (i, 0, 0)),
            pl.BlockSpec((1, 1, cblk), lambda i: (i, 0, 0)),
        ],
        out_shape=[
            jax.ShapeDtypeStruct((ntok, d, d), jnp.float32),
            jax.ShapeDtypeStruct((grid, 1, cblk), jnp.float32),
        ],
    )(tbg, eac, memp)
    return out, mem.reshape(c)


def kernel(tokens, word2state, state_emb, next_state_emb, projection,
           start_emb):
    n, t = tokens.shape
    c, h = state_emb.shape
    s = word2state.shape[1]
    d = projection.shape[1]
    ntok = n * t

    tbx, ap = _precompute(state_emb, next_state_emb, projection, start_emb)

    tok_flat = tokens.reshape(ntok).astype(jnp.int32)
    w2s = word2state.astype(jnp.int32)

    sc_gather = _make_sc_gather(ntok, s, d, c)
    tbg, eac, memp = sc_gather(tok_flat, w2s, tbx, ap)

    out, mem = _logmvv(tbg, eac, memp, ntok, s, d, c)
    return out.reshape(n, t, d, d), mem


# speedup vs baseline: 2.8858x; 1.0584x over previous
import functools

import jax
import jax.numpy as jnp
from jax import lax
from jax.experimental import pallas as pl
from jax.experimental.pallas import tpu as pltpu
from jax.experimental.pallas import tpu_sc as plsc

_NC = 2
_NS = 16
_NW = _NC * _NS
_LANES = 16


def _log_softmax(x):
    m = jnp.max(x, axis=-1, keepdims=True)
    s = jnp.sum(jnp.exp(x - m), axis=-1, keepdims=True)
    return x - m - jnp.log(s)


def _precompute_body(se_ref, ne_ref, proj_ref, start_ref, tbx_ref, ap_ref):
    se = se_ref[...]
    ne = ne_ref[...]
    proj = proj_ref[...]
    start = start_ref[...]
    bp = _log_softmax(jnp.dot(se, proj, preferred_element_type=jnp.float32))
    cp = _log_softmax(jnp.dot(ne, proj, preferred_element_type=jnp.float32))
    tbx_ref[...] = jnp.concatenate([bp, cp], axis=-1)
    ap = lax.dot_general(se, start,
                         dimension_numbers=(((1,), (1,)), ((), ())),
                         preferred_element_type=jnp.float32)
    ap_ref[...] = jnp.broadcast_to(ap, (ap.shape[0], _LANES))


def _precompute(state_emb, next_state_emb, projection, start_emb):
    c, h = state_emb.shape
    d = projection.shape[1]
    blk = 2048
    grid = c // blk
    return pl.pallas_call(
        _precompute_body,
        grid=(grid,),
        in_specs=[
            pl.BlockSpec((blk, h), lambda i: (i, 0)),
            pl.BlockSpec((blk, h), lambda i: (i, 0)),
            pl.BlockSpec((h, d), lambda i: (0, 0)),
            pl.BlockSpec((1, h), lambda i: (0, 0)),
        ],
        out_specs=[
            pl.BlockSpec((blk, 2 * d), lambda i: (i, 0)),
            pl.BlockSpec((blk, _LANES), lambda i: (i, 0)),
        ],
        out_shape=[
            jax.ShapeDtypeStruct((c, 2 * d), jnp.float32),
            jax.ShapeDtypeStruct((c, _LANES), jnp.float32),
        ],
    )(state_emb, next_state_emb, projection, start_emb.reshape(1, h))


def _make_sc_gather(ntok, s, d, c):
    tpw = ntok // _NW
    rows_pc = c // _NS
    mesh = plsc.VectorSubcoreMesh(core_axis_name="c", subcore_axis_name="s")

    @functools.partial(
        pl.kernel,
        out_type=(
            jax.ShapeDtypeStruct((ntok, s, 2 * d), jnp.float32),
            jax.ShapeDtypeStruct((ntok * s * _LANES // 128, 128),
                                 jnp.float32),
            jax.ShapeDtypeStruct((_NC, c * _LANES // 128, 128),
                                 jnp.float32),
        ),
        mesh=mesh,
        scratch_types=[
            pltpu.VMEM((tpw,), jnp.int32),
            pltpu.VMEM((tpw, s), jnp.int32),
            pltpu.VMEM((tpw, s, 2 * d), jnp.float32),
            pltpu.VMEM((tpw, s, _LANES), jnp.float32),
            pltpu.VMEM((tpw * s, _LANES), jnp.float32),
            pltpu.VMEM((tpw * s * _LANES // 128, 128), jnp.float32),
            pltpu.VMEM((256, _LANES), jnp.float32),
            pltpu.VMEM((rows_pc * _LANES // 128, 128), jnp.float32),
            pltpu.VMEM_SHARED((c, _LANES), jnp.float32),
            pltpu.SemaphoreType.DMA,
            pltpu.SemaphoreType.DMA,
        ],
        compiler_params=pltpu.CompilerParams(use_tc_tiling_on_sc=False),
    )
    def sc_gather(tok_hbm, w2s_hbm, tbx_hbm, ap_hbm,
                  tbg_out, eac_out, mem_out,
                  tok_v, st_v, tbg_v, ag_v, ea_v, eac_v, z_v, z3_v, mem_sh,
                  sem_b, sem_a):
        cid = lax.axis_index("c")
        sid = lax.axis_index("s")
        wid = sid * _NC + cid
        base = wid * tpw

        pltpu.sync_copy(tok_hbm.at[pl.ds(base, tpw)], tok_v)
        pltpu.async_copy(w2s_hbm.at[tok_v], st_v, sem_b).wait()

        descs = []
        ag_descs = []
        for i in range(tpw):
            descs.append(pltpu.async_copy(tbx_hbm.at[st_v.at[i]],
                                          tbg_v.at[i], sem_b))
        for i in range(tpw):
            ag_descs.append(pltpu.async_copy(ap_hbm.at[st_v.at[i]],
                                             ag_v.at[i], sem_a))

        zv = jnp.zeros((_LANES,), jnp.float32)

        def zbody(j, acc):
            for k in range(8):
                z_v[j * 8 + k] = zv
            return acc

        lax.fori_loop(0, 32, zbody, jnp.int32(0))

        def zcopy(kk, acc):
            pltpu.sync_copy(
                z_v, mem_sh.at[pl.ds(sid * rows_pc + kk * 256, 256)])
            return acc

        lax.fori_loop(0, rows_pc // 256, zcopy, jnp.int32(0))

        for de in ag_descs:
            de.wait()

        neg_inf = jnp.full((_LANES,), -jnp.inf, jnp.float32)
        unr = 128 // _LANES
        for i in range(tpw):
            def mbody(q, acc, i=i):
                for k in range(unr):
                    acc = jnp.maximum(acc, ag_v[i, q * unr + k])
                return acc

            m = lax.fori_loop(0, s // unr, mbody, neg_inf)

            def zsum(q, acc, i=i, m=m):
                for k in range(unr):
                    eu = jnp.exp(ag_v[i, q * unr + k] - m)
                    ea_v[i * s + q * unr + k] = eu
                    acc = acc + eu
                return acc

            z = lax.fori_loop(0, s // unr, zsum,
                              jnp.zeros((_LANES,), jnp.float32))
            rz = jnp.float32(1.0) / z

            def spack(q, acc, i=i, rz=rz):
                for k in range(unr):
                    v = ea_v[i * s + q * unr + k] * rz
                    ea_v[i * s + q * unr + k] = v
                    eac_v[i * (s // unr) + q, pl.ds(k * _LANES, _LANES)] = v
                return acc

            lax.fori_loop(0, s // unr, spack, jnp.int32(0))

        for de in descs:
            de.wait()
        wdesc = pltpu.async_copy(tbg_v, tbg_out.at[pl.ds(base, tpw)], sem_b)
        erows = tpw * s * _LANES // 128
        pltpu.sync_copy(eac_v, eac_out.at[pl.ds(wid * erows, erows)])
        wdesc.wait()

        plsc.subcore_barrier()
        sdescs = []
        for i in range(tpw):
            sdescs.append(pltpu.async_copy(ea_v.at[pl.ds(i * s, s)],
                                           mem_sh.at[st_v.at[i]], sem_a,
                                           add=True))
        for de in sdescs:
            de.wait()
        plsc.subcore_barrier()

        rows_pt = rows_pc * _LANES // 128

        def dchunk(kk, acc):
            pltpu.sync_copy(
                mem_sh.at[pl.ds(sid * rows_pc + kk * 256, 256)], z_v)

            def rbody(q, acc2, kk=kk):
                for k in range(128 // _LANES):
                    z3_v[kk * 32 + q, pl.ds(k * _LANES, _LANES)] = \
                        z_v[q * 8 + k]
                return acc2

            lax.fori_loop(0, 32, rbody, jnp.int32(0))
            return acc

        lax.fori_loop(0, rows_pc // 256, dchunk, jnp.int32(0))

        @pl.when(cid == 0)
        def _():
            pltpu.sync_copy(z3_v, mem_out.at[0].at[pl.ds(sid * rows_pt,
                                                         rows_pt)])

        @pl.when(cid == 1)
        def _():
            pltpu.sync_copy(z3_v, mem_out.at[1].at[pl.ds(sid * rows_pt,
                                                         rows_pt)])

    return sc_gather


def _logmvv_body(s, tbg_ref, eac_ref, memp_ref, out_ref, mem_ref):
    tb, _, d2 = tbg_ref.shape
    d = d2 // 2
    tbg = tbg_ref[...]
    ez = eac_ref[...]
    ea = ez.reshape(ez.shape[0], 128 // _LANES,
                    _LANES)[:, :, 0].reshape(tb, s)
    a = jnp.log(ea)
    lane = lax.broadcasted_iota(jnp.int32, (tb, s, d2), 2)
    a3 = jnp.broadcast_to(a[:, :, None], (tb, s, d2))
    abg = tbg + jnp.where(lane < d, a3, jnp.float32(0.0))
    m = jnp.max(abg, axis=1, keepdims=True)
    e = jnp.exp(abg - m)
    w = e[:, :, :d]
    ec = e[:, :, d:]
    sm = lax.dot_general(w, ec,
                         dimension_numbers=(((1,), (1,)), ((0,), (0,))),
                         preferred_element_type=jnp.float32)
    out_ref[...] = (m[:, 0, :d].reshape(tb, d, 1)
                    + m[:, 0, d:].reshape(tb, 1, d) + jnp.log(sm + 1e-30))
    mp = memp_ref[0] + memp_ref[1]
    r = mp.shape[0]
    lane0 = mp.reshape(r, 128 // _LANES, _LANES)[:, :, 0]
    mem_ref[...] = lane0.reshape(1, 1, r * (128 // _LANES))


def _logmvv(tbg, eac, memp, ntok, s, d, c):
    tb = 32
    grid = ntok // tb
    cblk = c // grid
    rblk = cblk * _LANES // 128
    out, mem = pl.pallas_call(
        functools.partial(_logmvv_body, s),
        grid=(grid,),
        in_specs=[
            pl.BlockSpec((tb, s, 2 * d), lambda i: (i, 0, 0)),
            pl.BlockSpec((tb * s * _LANES // 128, 128), lambda i: (i, 0)),
            pl.BlockSpec((_NC, rblk, 128), lambda i: (0, i, 0)),
        ],
        out_specs=[
            pl.BlockSpec((tb, d, d), lambda i: (i, 0, 0)),
            pl.BlockSpec((1, 1, cblk), lambda i: (i, 0, 0)),
        ],
        out_shape=[
            jax.ShapeDtypeStruct((ntok, d, d), jnp.float32),
            jax.ShapeDtypeStruct((grid, 1, cblk), jnp.float32),
        ],
    )(tbg, eac, memp)
    return out, mem.reshape(c)


def kernel(tokens, word2state, state_emb, next_state_emb, projection,
           start_emb):
    n, t = tokens.shape
    c, h = state_emb.shape
    s = word2state.shape[1]
    d = projection.shape[1]
    ntok = n * t

    tbx, ap = _precompute(state_emb, next_state_emb, projection, start_emb)

    tok_flat = tokens.reshape(ntok).astype(jnp.int32)
    w2s = word2state.astype(jnp.int32)

    sc_gather = _make_sc_gather(ntok, s, d, c)
    tbg, eac, memp = sc_gather(tok_flat, w2s, tbx, ap)

    out, mem = _logmvv(tbg, eac, memp, ntok, s, d, c)
    return out.reshape(n, t, d, d), mem
